# Initial kernel scaffold; baseline (speedup 1.0000x reference)
import functools

import jax
import jax.numpy as jnp
from jax import lax
from jax.experimental import pallas as pl
from jax.experimental.pallas import tpu as pltpu
from jax.experimental.pallas import tpu_sc as plsc

_VOCAB = 1000000
_EMBED = 64
_HIDDEN = 128
_CLS = 2
_B = 16384
_L = 200

_NC = 2
_NS = 16
_NW = _NC * _NS
_RPW = _B // _NW
_SR = 32
_CH = 2
_NCH = _SR // _CH
_NST = _RPW // _SR
_G = 100
_NG = _L // _G
_LANES = 16
_NV = _EMBED // _LANES


def _sc_pool_body(x_hbm, tbl_hbm, out_hbm, idx_v, rows_v, outb_v, sem0, sem1):
    c = lax.axis_index("c")
    s = lax.axis_index("s")
    wid = s * _NC + c
    base = wid * _RPW
    sems = (sem0, sem1)

    def fire(ch, b):
        for r in range(_CH):
            for j in range(_NG):
                pltpu.async_copy(
                    tbl_hbm.at[idx_v.at[ch * _CH + r, j]],
                    rows_v.at[b, pl.ds((r * _NG + j) * _G, _G)],
                    sems[b],
                )

    def wait(b):
        pltpu.make_async_copy(
            tbl_hbm.at[pl.ds(0, _CH * _L)], rows_v.at[b], sems[b]
        ).wait()

    def reduce_chunk(ch, b):
        for r in range(_CH):
            bufr = rows_v.at[b, pl.ds(r * _L, _L)]

            def rbody(q, acc):
                return tuple(
                    acc[v] + bufr[q, pl.ds(v * _LANES, _LANES)]
                    for v in range(_NV)
                )

            zero = jnp.zeros((_LANES,), jnp.float32)
            acc = lax.fori_loop(0, _L, rbody, (zero,) * _NV, unroll=8)
            row = ch * _CH + r
            for v in range(_NV):
                outb_v[row, pl.ds(v * _LANES, _LANES)] = acc[v]

    def stage_body(st, carry):
        row0 = base + st * _SR
        pltpu.sync_copy(x_hbm.at[pl.ds(row0, _SR)], idx_v)
        fire(0, 0)

        def pair_body(p, inner):
            ch0 = 2 * p
            fire(ch0 + 1, 1)
            wait(0)
            reduce_chunk(ch0, 0)

            @pl.when(ch0 + 2 < _NCH)
            def _():
                fire(ch0 + 2, 0)

            wait(1)
            reduce_chunk(ch0 + 1, 1)
            return inner

        lax.fori_loop(0, _NCH // 2, pair_body, 0)
        pltpu.sync_copy(outb_v, out_hbm.at[pl.ds(row0, _SR)])
        return carry

    lax.fori_loop(0, _NST, stage_body, 0)


_sc_pool = pl.kernel(
    _sc_pool_body,
    out_type=jax.ShapeDtypeStruct((_B, _EMBED), jnp.float32),
    mesh=plsc.VectorSubcoreMesh(core_axis_name="c", subcore_axis_name="s"),
    scratch_types=[
        pltpu.VMEM((_SR, _NG, _G), jnp.int32),
        pltpu.VMEM((2, _CH * _L, _EMBED), jnp.float32),
        pltpu.VMEM((_SR, _EMBED), jnp.float32),
        pltpu.SemaphoreType.DMA,
        pltpu.SemaphoreType.DMA,
    ],
)


def _mlp_body(p_ref, w1_ref, b1_ref, w2_ref, b2_ref, o_ref):
    p = p_ref[...] * (1.0 / _L)
    h = jnp.dot(p, w1_ref[...], preferred_element_type=jnp.float32) + b1_ref[...]
    h = jnp.maximum(h, 0.0)
    o_ref[...] = (
        jnp.dot(h, w2_ref[...], preferred_element_type=jnp.float32) + b2_ref[...]
    )


def _mlp(pooled, W1, b1, W2, b2):
    bb = 2048
    return pl.pallas_call(
        _mlp_body,
        grid=(_B // bb,),
        in_specs=[
            pl.BlockSpec((bb, _EMBED), lambda i: (i, 0)),
            pl.BlockSpec((_EMBED, _HIDDEN), lambda i: (0, 0)),
            pl.BlockSpec((1, _HIDDEN), lambda i: (0, 0)),
            pl.BlockSpec((_HIDDEN, _CLS), lambda i: (0, 0)),
            pl.BlockSpec((1, _CLS), lambda i: (0, 0)),
        ],
        out_specs=pl.BlockSpec((bb, _CLS), lambda i: (i, 0)),
        out_shape=jax.ShapeDtypeStruct((_B, _CLS), jnp.float32),
    )(pooled, W1, b1.reshape(1, _HIDDEN), W2, b2.reshape(1, _CLS))


def kernel(x, emb_table, W1, b1, W2, b2):
    xr = x.reshape(_B, _NG, _G)
    pooled = _sc_pool(xr, emb_table)
    return _mlp(pooled, W1, b1, W2, b2)


# speedup vs baseline: 3.0146x; 3.0146x over previous
import functools

import jax
import jax.numpy as jnp
from jax import lax
from jax.experimental import pallas as pl
from jax.experimental.pallas import tpu as pltpu
from jax.experimental.pallas import tpu_sc as plsc

_VOCAB = 1000000
_EMBED = 64
_HIDDEN = 128
_CLS = 2
_B = 16384
_L = 200

_NC = 2
_NS = 16
_NW = _NC * _NS
_RPW = _B // _NW
_SR = 32
_CH = 2
_NCH = _SR // _CH
_NST = _RPW // _SR
_G = 100
_NG = _L // _G
_LANES = 16
_NV = _EMBED // _LANES


def _sc_pool_body(x_hbm, tbl_hbm, out_hbm, idx_v, rows_v, outb_v, sem0, sem1):
    c = lax.axis_index("c")
    s = lax.axis_index("s")
    wid = s * _NC + c
    base = wid * _RPW
    sems = (sem0, sem1)

    def fire(ch, b):
        for r in range(_CH):
            for j in range(_NG):
                pltpu.async_copy(
                    tbl_hbm.at[idx_v.at[ch * _CH + r, j]],
                    rows_v.at[b, pl.ds((r * _NG + j) * _G, _G)],
                    sems[b],
                )

    def wait(b):
        pltpu.make_async_copy(
            tbl_hbm.at[pl.ds(0, _CH * _L)], rows_v.at[b], sems[b]
        ).wait()

    def reduce_chunk(ch, b):
        for r in range(_CH):
            bufr = rows_v.at[b, pl.ds(r * _L, _L)]

            def rbody(q, acc):
                return tuple(
                    acc[v] + bufr[q, pl.ds(v * _LANES, _LANES)]
                    for v in range(_NV)
                )

            zero = jnp.zeros((_LANES,), jnp.float32)
            acc = lax.fori_loop(0, _L, rbody, (zero,) * _NV, unroll=8)
            row = ch * _CH + r
            for v in range(_NV):
                outb_v[row, pl.ds(v * _LANES, _LANES)] = acc[v]

    def stage_body(st, carry):
        row0 = base + st * _SR
        pltpu.sync_copy(x_hbm.at[pl.ds(row0, _SR)], idx_v)
        fire(0, 0)

        def pair_body(p, inner):
            ch0 = 2 * p
            fire(ch0 + 1, 1)
            wait(0)
            reduce_chunk(ch0, 0)

            @pl.when(ch0 + 2 < _NCH)
            def _():
                fire(ch0 + 2, 0)

            wait(1)
            reduce_chunk(ch0 + 1, 1)
            return inner

        lax.fori_loop(0, _NCH // 2, pair_body, 0)
        pltpu.sync_copy(outb_v, out_hbm.at[pl.ds(row0, _SR)])
        return carry

    lax.fori_loop(0, _NST, stage_body, 0)


_sc_pool = pl.kernel(
    _sc_pool_body,
    out_type=jax.ShapeDtypeStruct((_B, _EMBED), jnp.float32),
    mesh=plsc.VectorSubcoreMesh(core_axis_name="c", subcore_axis_name="s"),
    scratch_types=[
        pltpu.VMEM((_SR, _NG, _G), jnp.int32),
        pltpu.VMEM((2, _CH * _L, _EMBED), jnp.float32),
        pltpu.VMEM((_SR, _EMBED), jnp.float32),
        pltpu.SemaphoreType.DMA,
        pltpu.SemaphoreType.DMA,
    ],
    compiler_params=pltpu.CompilerParams(use_tc_tiling_on_sc=False),
)


def _mlp_body(p_ref, w1_ref, b1_ref, w2_ref, b2_ref, o_ref):
    p = p_ref[...] * (1.0 / _L)
    h = jnp.dot(p, w1_ref[...], preferred_element_type=jnp.float32) + b1_ref[...]
    h = jnp.maximum(h, 0.0)
    o_ref[...] = (
        jnp.dot(h, w2_ref[...], preferred_element_type=jnp.float32) + b2_ref[...]
    )


def _mlp(pooled, W1, b1, W2, b2):
    bb = 2048
    return pl.pallas_call(
        _mlp_body,
        grid=(_B // bb,),
        in_specs=[
            pl.BlockSpec((bb, _EMBED), lambda i: (i, 0)),
            pl.BlockSpec((_EMBED, _HIDDEN), lambda i: (0, 0)),
            pl.BlockSpec((1, _HIDDEN), lambda i: (0, 0)),
            pl.BlockSpec((_HIDDEN, _CLS), lambda i: (0, 0)),
            pl.BlockSpec((1, _CLS), lambda i: (0, 0)),
        ],
        out_specs=pl.BlockSpec((bb, _CLS), lambda i: (i, 0)),
        out_shape=jax.ShapeDtypeStruct((_B, _CLS), jnp.float32),
    )(pooled, W1, b1.reshape(1, _HIDDEN), W2, b2.reshape(1, _CLS))


def kernel(x, emb_table, W1, b1, W2, b2):
    xr = x.reshape(_B, _NG, _G)
    pooled = _sc_pool(xr, emb_table)
    return _mlp(pooled, W1, b1, W2, b2)


# speedup vs baseline: 3.0598x; 1.0150x over previous
import functools

import jax
import jax.numpy as jnp
from jax import lax
from jax.experimental import pallas as pl
from jax.experimental.pallas import tpu as pltpu
from jax.experimental.pallas import tpu_sc as plsc

_VOCAB = 1000000
_EMBED = 64
_HIDDEN = 128
_CLS = 2
_B = 16384
_L = 200

_NC = 2
_NS = 16
_NW = _NC * _NS
_RPW = _B // _NW
_SR = 32
_CH = 2
_NCH = _SR // _CH
_NST = _RPW // _SR
_G_OFF = (0, 104)
_G_LEN = (104, 96)
_LANES = 16
_NV = _EMBED // _LANES


def _sc_pool_body(x_hbm, tbl_hbm, out_hbm, idx_v, rows_v, outb_v, sem0, sem1):
    c = lax.axis_index("c")
    s = lax.axis_index("s")
    wid = s * _NC + c
    base = wid * _RPW
    sems = (sem0, sem1)

    def fire(ch, b):
        for r in range(_CH):
            for off, ln in zip(_G_OFF, _G_LEN):
                pltpu.async_copy(
                    tbl_hbm.at[idx_v.at[ch * _CH + r, pl.ds(off, ln)]],
                    rows_v.at[b, pl.ds(r * _L + off, ln)],
                    sems[b],
                )

    def wait(b):
        pltpu.make_async_copy(
            tbl_hbm.at[pl.ds(0, _CH * _L)], rows_v.at[b], sems[b]
        ).wait()

    def reduce_chunk(ch, b):
        for r in range(_CH):
            bufr = rows_v.at[b, pl.ds(r * _L, _L)]

            def rbody(q, acc):
                return tuple(
                    acc[v] + bufr[q, pl.ds(v * _LANES, _LANES)]
                    for v in range(_NV)
                )

            zero = jnp.zeros((_LANES,), jnp.float32)
            acc = lax.fori_loop(0, _L, rbody, (zero,) * _NV, unroll=8)
            row = ch * _CH + r
            for v in range(_NV):
                outb_v[row, pl.ds(v * _LANES, _LANES)] = acc[v]

    def stage_body(st, carry):
        row0 = base + st * _SR
        pltpu.sync_copy(x_hbm.at[pl.ds(row0, _SR)], idx_v)
        fire(0, 0)

        def pair_body(p, inner):
            ch0 = 2 * p
            fire(ch0 + 1, 1)
            wait(0)
            reduce_chunk(ch0, 0)

            @pl.when(ch0 + 2 < _NCH)
            def _():
                fire(ch0 + 2, 0)

            wait(1)
            reduce_chunk(ch0 + 1, 1)
            return inner

        lax.fori_loop(0, _NCH // 2, pair_body, 0)
        pltpu.sync_copy(outb_v, out_hbm.at[pl.ds(row0, _SR)])
        return carry

    lax.fori_loop(0, _NST, stage_body, 0)


_sc_pool = pl.kernel(
    _sc_pool_body,
    out_type=jax.ShapeDtypeStruct((_B, _EMBED), jnp.float32),
    mesh=plsc.VectorSubcoreMesh(core_axis_name="c", subcore_axis_name="s"),
    scratch_types=[
        pltpu.VMEM((_SR, _L), jnp.int32),
        pltpu.VMEM((2, _CH * _L, _EMBED), jnp.float32),
        pltpu.VMEM((_SR, _EMBED), jnp.float32),
        pltpu.SemaphoreType.DMA,
        pltpu.SemaphoreType.DMA,
    ],
    compiler_params=pltpu.CompilerParams(use_tc_tiling_on_sc=False),
)


def _mlp_body(p_ref, w1_ref, b1_ref, w2_ref, b2_ref, o_ref):
    p = p_ref[...] * (1.0 / _L)
    h = jnp.dot(p, w1_ref[...], preferred_element_type=jnp.float32) + b1_ref[...]
    h = jnp.maximum(h, 0.0)
    o_ref[...] = (
        jnp.dot(h, w2_ref[...], preferred_element_type=jnp.float32) + b2_ref[...]
    )


def _mlp(pooled, W1, b1, W2, b2):
    bb = 2048
    return pl.pallas_call(
        _mlp_body,
        grid=(_B // bb,),
        in_specs=[
            pl.BlockSpec((bb, _EMBED), lambda i: (i, 0)),
            pl.BlockSpec((_EMBED, _HIDDEN), lambda i: (0, 0)),
            pl.BlockSpec((1, _HIDDEN), lambda i: (0, 0)),
            pl.BlockSpec((_HIDDEN, _CLS), lambda i: (0, 0)),
            pl.BlockSpec((1, _CLS), lambda i: (0, 0)),
        ],
        out_specs=pl.BlockSpec((bb, _CLS), lambda i: (i, 0)),
        out_shape=jax.ShapeDtypeStruct((_B, _CLS), jnp.float32),
    )(pooled, W1, b1.reshape(1, _HIDDEN), W2, b2.reshape(1, _CLS))


def kernel(x, emb_table, W1, b1, W2, b2):
    pooled = _sc_pool(x, emb_table)
    return _mlp(pooled, W1, b1, W2, b2)


# speedup vs baseline: 3.2114x; 1.0495x over previous
import functools

import jax
import jax.numpy as jnp
from jax import lax
from jax.experimental import pallas as pl
from jax.experimental.pallas import tpu as pltpu
from jax.experimental.pallas import tpu_sc as plsc

_VOCAB = 1000000
_EMBED = 64
_HIDDEN = 128
_CLS = 2
_B = 16384
_L = 200

_NC = 2
_NS = 16
_NW = _NC * _NS
_RPW = _B // _NW
_SR = 64
_CH = 4
_NCH = _SR // _CH
_NST = _RPW // _SR
_G_OFF = (0, 104)
_G_LEN = (104, 96)
_LANES = 16
_NV = _EMBED // _LANES


def _sc_pool_body(x_hbm, tbl_hbm, out_hbm, idx_v, rows_v, outb_v, sem0, sem1):
    c = lax.axis_index("c")
    s = lax.axis_index("s")
    wid = s * _NC + c
    base = wid * _RPW
    sems = (sem0, sem1)

    def fire(ch, b):
        for r in range(_CH):
            for off, ln in zip(_G_OFF, _G_LEN):
                pltpu.async_copy(
                    tbl_hbm.at[idx_v.at[ch * _CH + r, pl.ds(off, ln)]],
                    rows_v.at[b, pl.ds(r * _L + off, ln)],
                    sems[b],
                )

    def wait(b):
        pltpu.make_async_copy(
            tbl_hbm.at[pl.ds(0, _CH * _L)], rows_v.at[b], sems[b]
        ).wait()

    def reduce_chunk(ch, b):
        for r in range(_CH):
            bufr = rows_v.at[b, pl.ds(r * _L, _L)]

            def rbody(q, acc):
                return tuple(
                    acc[v] + bufr[q, pl.ds(v * _LANES, _LANES)]
                    for v in range(_NV)
                )

            zero = jnp.zeros((_LANES,), jnp.float32)
            acc = lax.fori_loop(0, _L, rbody, (zero,) * _NV, unroll=16)
            row = ch * _CH + r
            for v in range(_NV):
                outb_v[row, pl.ds(v * _LANES, _LANES)] = acc[v]

    def stage_body(st, carry):
        row0 = base + st * _SR
        pltpu.sync_copy(x_hbm.at[pl.ds(row0, _SR)], idx_v)
        fire(0, 0)

        def pair_body(p, inner):
            ch0 = 2 * p
            fire(ch0 + 1, 1)
            wait(0)
            reduce_chunk(ch0, 0)

            @pl.when(ch0 + 2 < _NCH)
            def _():
                fire(ch0 + 2, 0)

            wait(1)
            reduce_chunk(ch0 + 1, 1)
            return inner

        lax.fori_loop(0, _NCH // 2, pair_body, 0)
        pltpu.sync_copy(outb_v, out_hbm.at[pl.ds(row0, _SR)])
        return carry

    lax.fori_loop(0, _NST, stage_body, 0)


_sc_pool = pl.kernel(
    _sc_pool_body,
    out_type=jax.ShapeDtypeStruct((_B, _EMBED), jnp.float32),
    mesh=plsc.VectorSubcoreMesh(core_axis_name="c", subcore_axis_name="s"),
    scratch_types=[
        pltpu.VMEM((_SR, _L), jnp.int32),
        pltpu.VMEM((2, _CH * _L, _EMBED), jnp.float32),
        pltpu.VMEM((_SR, _EMBED), jnp.float32),
        pltpu.SemaphoreType.DMA,
        pltpu.SemaphoreType.DMA,
    ],
    compiler_params=pltpu.CompilerParams(use_tc_tiling_on_sc=False),
)


def _mlp_body(p_ref, w1_ref, b1_ref, w2_ref, b2_ref, o_ref):
    p = p_ref[...] * (1.0 / _L)
    h = jnp.dot(p, w1_ref[...], preferred_element_type=jnp.float32) + b1_ref[...]
    h = jnp.maximum(h, 0.0)
    o_ref[...] = (
        jnp.dot(h, w2_ref[...], preferred_element_type=jnp.float32) + b2_ref[...]
    )


def _mlp(pooled, W1, b1, W2, b2):
    bb = 2048
    return pl.pallas_call(
        _mlp_body,
        grid=(_B // bb,),
        in_specs=[
            pl.BlockSpec((bb, _EMBED), lambda i: (i, 0)),
            pl.BlockSpec((_EMBED, _HIDDEN), lambda i: (0, 0)),
            pl.BlockSpec((1, _HIDDEN), lambda i: (0, 0)),
            pl.BlockSpec((_HIDDEN, _CLS), lambda i: (0, 0)),
            pl.BlockSpec((1, _CLS), lambda i: (0, 0)),
        ],
        out_specs=pl.BlockSpec((bb, _CLS), lambda i: (i, 0)),
        out_shape=jax.ShapeDtypeStruct((_B, _CLS), jnp.float32),
    )(pooled, W1, b1.reshape(1, _HIDDEN), W2, b2.reshape(1, _CLS))


def kernel(x, emb_table, W1, b1, W2, b2):
    pooled = _sc_pool(x, emb_table)
    return _mlp(pooled, W1, b1, W2, b2)
